# baseline (device time: 107601 ns/iter reference)
import jax
import jax.numpy as jnp
from jax import lax
from jax.experimental import pallas as pl
from jax.experimental.pallas import tpu as pltpu

N_DEV = 32
PLANE = 8
NZ = 4
SEG1 = 2


def kernel(x, w_mat):
    m, k_per = x.shape
    _, n = w_mat.shape
    m_per = m // N_DEV
    half = n // 2

    def body(x_ref, w_ref, out_ref, part_ref, rbuf1, rbuf2,
             s1_send, s1_recv, s2_send, s2_recv):
        p = lax.axis_index("i")
        z = lax.div(p, PLANE)
        q = lax.rem(p, PLANE)

        def q_to_r(qq):
            yy = lax.div(qq, 2)
            xx = lax.rem(qq + yy, 2)
            return jnp.where(xx == 1, 1 + yy, lax.rem(8 - yy, 8))

        def r_to_q(rr):
            xx = jnp.where((rr >= 1) & (rr <= 4), 1, 0)
            yy = jnp.where(xx == 1, rr - 1, lax.rem(8 - rr, 8))
            return 2 * yy + lax.rem(xx + yy, 2)

        r = q_to_r(q)

        succ = z * PLANE + r_to_q(lax.rem(r + 1, 8))
        pred = z * PLANE + r_to_q(lax.rem(r + 7, 8))
        up = lax.rem(z + 1, NZ) * PLANE + q
        down = lax.rem(z + 3, NZ) * PLANE + q

        part_ref[:, :] = jnp.dot(
            x_ref[:, :], w_ref[:, :], preferred_element_type=jnp.float32
        )

        barrier_sem = pltpu.get_barrier_semaphore()
        for nbr in (succ, pred, up, down):
            pl.semaphore_signal(
                barrier_sem, inc=1,
                device_id=(nbr,), device_id_type=pl.DeviceIdType.MESH,
            )
        pl.semaphore_wait(barrier_sem, 4)

        col0 = (0, half)
        sign = (-1, 1)
        dst1 = (succ, pred)
        dst2 = (up, down)

        def pref(b, dirn):
            return part_ref.at[pl.ds(b * m_per, m_per),
                               pl.ds(col0[dirn], half)]

        def pval(b, dirn):
            return part_ref[pl.ds(b * m_per, m_per),
                            pl.ds(col0[dirn], half)]

        rdmas = {}

        def zb_of(dirn, g):
            if dirn == 0:
                return lax.rem(z + 3 - g + NZ, NZ)
            return lax.rem(z + 1 + g, NZ)

        segw = half // SEG1

        def p1_make(dirn, h, g, s):
            cds = pl.ds(s * segw, segw)
            if h == 0:
                rc = lax.rem(r + sign[dirn] + 16, 8)
                b = zb_of(dirn, g) * PLANE + r_to_q(rc)
                src = part_ref.at[pl.ds(b * m_per, m_per),
                                  pl.ds(col0[dirn] + s * segw, segw)]
            else:
                src = rbuf1.at[dirn, h - 1, g, :, cds]
            return pltpu.make_async_remote_copy(
                src_ref=src,
                dst_ref=rbuf1.at[dirn, h, g, :, cds],
                send_sem=s1_send.at[dirn, h, g, s],
                recv_sem=s1_recv.at[dirn, h, g, s],
                device_id=(dst1[dirn],),
                device_id_type=pl.DeviceIdType.MESH,
            )

        def p2_make(dirn, h):
            if h == 0:
                src = rbuf1.at[dirn, PLANE - 2, 0]
            else:
                src = rbuf2.at[dirn, h - 1]
            return pltpu.make_async_remote_copy(
                src_ref=src,
                dst_ref=rbuf2.at[dirn, h],
                send_sem=s2_send.at[dirn, h],
                recv_sem=s2_recv.at[dirn, h],
                device_id=(dst2[dirn],),
                device_id_type=pl.DeviceIdType.MESH,
            )

        for g in range(NZ):
            for s in range(SEG1):
                for dirn in (0, 1):
                    rd = p1_make(dirn, 0, g, s)
                    rd.start()
                    rdmas[(1, dirn, 0, g, s)] = rd

        for h in range(PLANE - 1):
            for g in range(NZ):
                for s in range(SEG1):
                    for dirn in (0, 1):
                        rdmas[(1, dirn, h, g, s)].wait_recv()
                        rc = lax.rem(r + sign[dirn] * (2 + h) + 32, 8)
                        b = zb_of(dirn, g) * PLANE + r_to_q(rc)
                        cds = pl.ds(s * segw, segw)
                        rbuf1[dirn, h, g, :, cds] = (
                            rbuf1[dirn, h, g, :, cds]
                            + part_ref[pl.ds(b * m_per, m_per),
                                       pl.ds(col0[dirn] + s * segw, segw)]
                        )
                        if h < PLANE - 2:
                            rd = p1_make(dirn, h + 1, g, s)
                            rd.start()
                            rdmas[(1, dirn, h + 1, g, s)] = rd
                for dirn in (0, 1):
                    if h < PLANE - 2:
                        continue
                    if True:
                        if g == 0:
                            rd = p2_make(dirn, 0)
                            rd.start()
                            rdmas[(2, dirn, 0, 0)] = rd
                        elif g < NZ - 1:
                            rdmas[(2, dirn, g - 1, 0)].wait_recv()
                            rbuf2[dirn, g - 1] = (
                                rbuf2[dirn, g - 1] + rbuf1[dirn, PLANE - 2, g]
                            )
                            rd = p2_make(dirn, g)
                            rd.start()
                            rdmas[(2, dirn, g, 0)] = rd
                        else:
                            rdmas[(2, dirn, g - 1, 0)].wait_recv()
                            y = (
                                rbuf2[dirn, g - 1]
                                + rbuf1[dirn, PLANE - 2, g]
                            )
                            yc = jnp.clip(y, -60.0, 60.0)
                            out_ref[:, pl.ds(col0[dirn], half)] = (
                                y / (1.0 + jnp.exp(-yc))
                            )

        for key in rdmas:
            rdmas[key].wait_send()

    return pl.pallas_call(
        body,
        out_shape=jax.ShapeDtypeStruct((m_per, n), jnp.float32),
        in_specs=[
            pl.BlockSpec(memory_space=pltpu.VMEM),
            pl.BlockSpec(memory_space=pltpu.VMEM),
        ],
        out_specs=pl.BlockSpec(memory_space=pltpu.VMEM),
        scratch_shapes=[
            pltpu.VMEM((m, n), jnp.float32),
            pltpu.VMEM((2, PLANE - 1, NZ, m_per, half), jnp.float32),
            pltpu.VMEM((2, NZ - 1, m_per, half), jnp.float32),
            pltpu.SemaphoreType.DMA((2, PLANE - 1, NZ, SEG1)),
            pltpu.SemaphoreType.DMA((2, PLANE - 1, NZ, SEG1)),
            pltpu.SemaphoreType.DMA((2, NZ - 1)),
            pltpu.SemaphoreType.DMA((2, NZ - 1)),
        ],
        compiler_params=pltpu.CompilerParams(
            collective_id=0, vmem_limit_bytes=64 * 1024 * 1024
        ),
    )(x, w_mat)


# device time: 106663 ns/iter; 1.0088x vs baseline; 1.0088x over previous
import jax
import jax.numpy as jnp
from jax import lax
from jax.experimental import pallas as pl
from jax.experimental.pallas import tpu as pltpu

N_DEV = 32
PLANE = 8
NZ = 4
ACOL = 1280
BZF = 4


def kernel(x, w_mat):
    m, k_per = x.shape
    _, n = w_mat.shape
    m_per = m // N_DEV
    ahalf = ACOL // 2
    bcol = n - ACOL
    bhalf = bcol // 2
    mz = m // NZ
    bzw = mz // BZF

    def body(x_ref, w_ref, out_ref, part_ref, rbufA1, rbufA2, rbufB1,
             rbufB2, sA1_s, sA1_r, sA2_s, sA2_r, sB1_s, sB1_r,
             sB2_s, sB2_r):
        p = lax.axis_index("i")
        z = lax.div(p, PLANE)
        q = lax.rem(p, PLANE)

        def q_to_r(qq):
            yy = lax.div(qq, 2)
            xx = lax.rem(qq + yy, 2)
            return jnp.where(xx == 1, 1 + yy, lax.rem(8 - yy, 8))

        def r_to_q(rr):
            xx = jnp.where((rr >= 1) & (rr <= 4), 1, 0)
            yy = jnp.where(xx == 1, rr - 1, lax.rem(8 - rr, 8))
            return 2 * yy + lax.rem(xx + yy, 2)

        r = q_to_r(q)

        succ = z * PLANE + r_to_q(lax.rem(r + 1, 8))
        pred = z * PLANE + r_to_q(lax.rem(r + 7, 8))
        up = lax.rem(z + 1, NZ) * PLANE + q
        down = lax.rem(z + 3, NZ) * PLANE + q

        part_ref[:, :] = jnp.dot(
            x_ref[:, :], w_ref[:, :], preferred_element_type=jnp.float32
        )

        barrier_sem = pltpu.get_barrier_semaphore()
        for nbr in (succ, pred, up, down):
            pl.semaphore_signal(
                barrier_sem, inc=1,
                device_id=(nbr,), device_id_type=pl.DeviceIdType.MESH,
            )
        pl.semaphore_wait(barrier_sem, 4)

        acol0 = (0, ahalf)
        bcol0 = (ACOL, ACOL + bhalf)
        sign = (-1, 1)
        dst_pl = (succ, pred)
        dst_z = (up, down)

        rdmas = {}

        def zb_of(dirn, g):
            if dirn == 0:
                return lax.rem(z + 3 - g + NZ, NZ)
            return lax.rem(z + 1 + g, NZ)

        def a1_make(dirn, h, g):
            if h == 0:
                rc = lax.rem(r + sign[dirn] + 16, 8)
                b = zb_of(dirn, g) * PLANE + r_to_q(rc)
                src = part_ref.at[pl.ds(b * m_per, m_per),
                                  pl.ds(acol0[dirn], ahalf)]
            else:
                src = rbufA1.at[dirn, h - 1, g]
            return pltpu.make_async_remote_copy(
                src_ref=src,
                dst_ref=rbufA1.at[dirn, h, g],
                send_sem=sA1_s.at[dirn, h, g],
                recv_sem=sA1_r.at[dirn, h, g],
                device_id=(dst_pl[dirn],),
                device_id_type=pl.DeviceIdType.MESH,
            )

        def a2_make(dirn, h):
            if h == 0:
                src = rbufA1.at[dirn, PLANE - 2, 0]
            else:
                src = rbufA2.at[dirn, h - 1]
            return pltpu.make_async_remote_copy(
                src_ref=src,
                dst_ref=rbufA2.at[dirn, h],
                send_sem=sA2_s.at[dirn, h],
                recv_sem=sA2_r.at[dirn, h],
                device_id=(dst_z[dirn],),
                device_id_type=pl.DeviceIdType.MESH,
            )

        def b1_make(dirn, h, s):
            rds = pl.ds(s * bzw, bzw)
            if h == 0:
                zc = lax.rem(z + sign[dirn] + 8, NZ)
                src = part_ref.at[pl.ds(zc * mz + s * bzw, bzw),
                                  pl.ds(bcol0[dirn], bhalf)]
            else:
                src = rbufB1.at[dirn, h - 1, rds]
            return pltpu.make_async_remote_copy(
                src_ref=src,
                dst_ref=rbufB1.at[dirn, h, rds],
                send_sem=sB1_s.at[dirn, h, s],
                recv_sem=sB1_r.at[dirn, h, s],
                device_id=(dst_z[dirn],),
                device_id_type=pl.DeviceIdType.MESH,
            )

        def b1_process(h):
            for s in range(BZF):
                for dirn in (0, 1):
                    rdmas[(3, dirn, h, s)].wait_recv()
                    zc = lax.rem(z + sign[dirn] * (2 + h) + 8, NZ)
                    rds = pl.ds(s * bzw, bzw)
                    rbufB1[dirn, h, rds] = (
                        rbufB1[dirn, h, rds]
                        + part_ref[pl.ds(zc * mz + s * bzw, bzw),
                                   pl.ds(bcol0[dirn], bhalf)]
                    )
                    if h < NZ - 2:
                        rd = b1_make(dirn, h + 1, s)
                        rd.start()
                        rdmas[(3, dirn, h + 1, s)] = rd

        def b2_make(dirn, h):
            if h == 0:
                rc = lax.rem(r + sign[dirn] + 16, 8)
                src = rbufB1.at[dirn, NZ - 2,
                                pl.ds(r_to_q(rc) * m_per, m_per)]
            else:
                src = rbufB2.at[dirn, h - 1]
            return pltpu.make_async_remote_copy(
                src_ref=src,
                dst_ref=rbufB2.at[dirn, h],
                send_sem=sB2_s.at[dirn, h],
                recv_sem=sB2_r.at[dirn, h],
                device_id=(dst_pl[dirn],),
                device_id_type=pl.DeviceIdType.MESH,
            )

        for g in range(NZ):
            for dirn in (0, 1):
                rd = a1_make(dirn, 0, g)
                rd.start()
                rdmas[(1, dirn, 0, g)] = rd
        for s in range(BZF):
            for dirn in (0, 1):
                rd = b1_make(dirn, 0, s)
                rd.start()
                rdmas[(3, dirn, 0, s)] = rd

        for h in range(PLANE - 1):
            for g in range(NZ):
                for dirn in (0, 1):
                    rdmas[(1, dirn, h, g)].wait_recv()
                    rc = lax.rem(r + sign[dirn] * (2 + h) + 32, 8)
                    b = zb_of(dirn, g) * PLANE + r_to_q(rc)
                    rbufA1[dirn, h, g] = (
                        rbufA1[dirn, h, g]
                        + part_ref[pl.ds(b * m_per, m_per),
                                   pl.ds(acol0[dirn], ahalf)]
                    )
                    if h < PLANE - 2:
                        rd = a1_make(dirn, h + 1, g)
                        rd.start()
                        rdmas[(1, dirn, h + 1, g)] = rd
                    else:
                        if g == 0:
                            rd = a2_make(dirn, 0)
                            rd.start()
                            rdmas[(2, dirn, 0)] = rd
                        elif g < NZ - 1:
                            rdmas[(2, dirn, g - 1)].wait_recv()
                            rbufA2[dirn, g - 1] = (
                                rbufA2[dirn, g - 1]
                                + rbufA1[dirn, PLANE - 2, g]
                            )
                            rd = a2_make(dirn, g)
                            rd.start()
                            rdmas[(2, dirn, g)] = rd
                        else:
                            rdmas[(2, dirn, g - 1)].wait_recv()
                            y = (
                                rbufA2[dirn, g - 1]
                                + rbufA1[dirn, PLANE - 2, g]
                            )
                            yc = jnp.clip(y, -60.0, 60.0)
                            out_ref[:, pl.ds(acol0[dirn], ahalf)] = (
                                y / (1.0 + jnp.exp(-yc))
                            )
            if h == 2:
                b1_process(0)
            elif h == 4:
                b1_process(1)

        b1_process(2)

        for dirn in (0, 1):
            rd = b2_make(dirn, 0)
            rd.start()
            rdmas[(4, dirn, 0)] = rd
        for h in range(PLANE - 1):
            for dirn in (0, 1):
                rdmas[(4, dirn, h)].wait_recv()
                rc = lax.rem(r + sign[dirn] * (2 + h) + 32, 8)
                qc = r_to_q(rc)
                if h < PLANE - 2:
                    rbufB2[dirn, h] = (
                        rbufB2[dirn, h]
                        + rbufB1[dirn, NZ - 2, pl.ds(qc * m_per, m_per)]
                    )
                    rd = b2_make(dirn, h + 1)
                    rd.start()
                    rdmas[(4, dirn, h + 1)] = rd
                else:
                    y = (
                        rbufB2[dirn, h]
                        + rbufB1[dirn, NZ - 2, pl.ds(qc * m_per, m_per)]
                    )
                    yc = jnp.clip(y, -60.0, 60.0)
                    out_ref[:, pl.ds(bcol0[dirn], bhalf)] = (
                        y / (1.0 + jnp.exp(-yc))
                    )

        for key in rdmas:
            rdmas[key].wait_send()

    return pl.pallas_call(
        body,
        out_shape=jax.ShapeDtypeStruct((m_per, n), jnp.float32),
        in_specs=[
            pl.BlockSpec(memory_space=pltpu.VMEM),
            pl.BlockSpec(memory_space=pltpu.VMEM),
        ],
        out_specs=pl.BlockSpec(memory_space=pltpu.VMEM),
        scratch_shapes=[
            pltpu.VMEM((m, n), jnp.float32),
            pltpu.VMEM((2, PLANE - 1, NZ, m_per, ahalf), jnp.float32),
            pltpu.VMEM((2, NZ - 1, m_per, ahalf), jnp.float32),
            pltpu.VMEM((2, NZ - 1, mz, bhalf), jnp.float32),
            pltpu.VMEM((2, PLANE - 1, m_per, bhalf), jnp.float32),
            pltpu.SemaphoreType.DMA((2, PLANE - 1, NZ)),
            pltpu.SemaphoreType.DMA((2, PLANE - 1, NZ)),
            pltpu.SemaphoreType.DMA((2, NZ - 1)),
            pltpu.SemaphoreType.DMA((2, NZ - 1)),
            pltpu.SemaphoreType.DMA((2, NZ - 1, BZF)),
            pltpu.SemaphoreType.DMA((2, NZ - 1, BZF)),
            pltpu.SemaphoreType.DMA((2, PLANE - 1)),
            pltpu.SemaphoreType.DMA((2, PLANE - 1)),
        ],
        compiler_params=pltpu.CompilerParams(
            collective_id=0, vmem_limit_bytes=64 * 1024 * 1024
        ),
    )(x, w_mat)


# device time: 98414 ns/iter; 1.0934x vs baseline; 1.0838x over previous
import jax
import jax.numpy as jnp
from jax import lax
from jax.experimental import pallas as pl
from jax.experimental.pallas import tpu as pltpu

N_DEV = 32
PLANE = 8
NZ = 4
ACOL = 1280
BZF = 4


def kernel(x, w_mat):
    m, k_per = x.shape
    _, n = w_mat.shape
    m_per = m // N_DEV
    ahalf = ACOL // 2
    bcol = n - ACOL
    bhalf = bcol // 2
    mz = m // NZ
    bzw = mz // BZF

    def body(x_ref, w_ref, out_ref, part_ref, rbufA1, rbufA2, rbufB1,
             rbufB2, sA1_s, sA1_r, sA2_s, sA2_r, sB1_s, sB1_r,
             sB2_s, sB2_r):
        p = lax.axis_index("i")
        z = lax.div(p, PLANE)
        q = lax.rem(p, PLANE)

        def q_to_r(qq):
            yy = lax.div(qq, 2)
            xx = lax.rem(qq + yy, 2)
            return jnp.where(xx == 1, 1 + yy, lax.rem(8 - yy, 8))

        def r_to_q(rr):
            xx = jnp.where((rr >= 1) & (rr <= 4), 1, 0)
            yy = jnp.where(xx == 1, rr - 1, lax.rem(8 - rr, 8))
            return 2 * yy + lax.rem(xx + yy, 2)

        r = q_to_r(q)

        succ = z * PLANE + r_to_q(lax.rem(r + 1, 8))
        pred = z * PLANE + r_to_q(lax.rem(r + 7, 8))
        up = lax.rem(z + 1, NZ) * PLANE + q
        down = lax.rem(z + 3, NZ) * PLANE + q

        part_ref[:, :] = jnp.dot(
            x_ref[:, :], w_ref[:, :], preferred_element_type=jnp.float32
        )

        barrier_sem = pltpu.get_barrier_semaphore()
        for nbr in (succ, pred, up, down):
            pl.semaphore_signal(
                barrier_sem, inc=1,
                device_id=(nbr,), device_id_type=pl.DeviceIdType.MESH,
            )
        pl.semaphore_wait(barrier_sem, 4)

        acol0 = (0, ahalf)
        bcol0 = (ACOL, ACOL + bhalf)
        sign = (-1, 1)
        dst_pl = (succ, pred)
        dst_z = (up, down)

        rdmas = {}

        def zb_of(dirn, g):
            if dirn == 0:
                return lax.rem(z + 3 - g + NZ, NZ)
            return lax.rem(z + 1 + g, NZ)

        def a1_make(dirn, h, g):
            if h == 0:
                rc = lax.rem(r + sign[dirn] + 16, 8)
                b = zb_of(dirn, g) * PLANE + r_to_q(rc)
                src = part_ref.at[pl.ds(b * m_per, m_per),
                                  pl.ds(acol0[dirn], ahalf)]
            else:
                src = rbufA1.at[dirn, h - 1, g]
            return pltpu.make_async_remote_copy(
                src_ref=src,
                dst_ref=rbufA1.at[dirn, h, g],
                send_sem=sA1_s.at[dirn, h, g],
                recv_sem=sA1_r.at[dirn, h, g],
                device_id=(dst_pl[dirn],),
                device_id_type=pl.DeviceIdType.MESH,
            )

        def a2_make(dirn, h):
            if h == 0:
                src = rbufA1.at[dirn, PLANE - 2, 0]
            else:
                src = rbufA2.at[dirn, h - 1]
            return pltpu.make_async_remote_copy(
                src_ref=src,
                dst_ref=rbufA2.at[dirn, h],
                send_sem=sA2_s.at[dirn, h],
                recv_sem=sA2_r.at[dirn, h],
                device_id=(dst_z[dirn],),
                device_id_type=pl.DeviceIdType.MESH,
            )

        def b1_make(dirn, h, s):
            rds = pl.ds(s * bzw, bzw)
            if h == 0:
                zc = lax.rem(z + sign[dirn] + 8, NZ)
                src = part_ref.at[pl.ds(zc * mz + s * bzw, bzw),
                                  pl.ds(bcol0[dirn], bhalf)]
            else:
                src = rbufB1.at[dirn, h - 1, rds]
            return pltpu.make_async_remote_copy(
                src_ref=src,
                dst_ref=rbufB1.at[dirn, h, rds],
                send_sem=sB1_s.at[dirn, h, s],
                recv_sem=sB1_r.at[dirn, h, s],
                device_id=(dst_z[dirn],),
                device_id_type=pl.DeviceIdType.MESH,
            )

        def b1_process(h):
            for s in range(BZF):
                for dirn in (0, 1):
                    rdmas[(3, dirn, h, s)].wait_recv()
                    zc = lax.rem(z + sign[dirn] * (2 + h) + 8, NZ)
                    rds = pl.ds(s * bzw, bzw)
                    rbufB1[dirn, h, rds] = (
                        rbufB1[dirn, h, rds]
                        + part_ref[pl.ds(zc * mz + s * bzw, bzw),
                                   pl.ds(bcol0[dirn], bhalf)]
                    )
                    if h < NZ - 2:
                        rd = b1_make(dirn, h + 1, s)
                        rd.start()
                        rdmas[(3, dirn, h + 1, s)] = rd

        def b2_make(dirn, h):
            if h == 0:
                rc = lax.rem(r + sign[dirn] + 16, 8)
                src = rbufB1.at[dirn, NZ - 2,
                                pl.ds(r_to_q(rc) * m_per, m_per)]
            else:
                src = rbufB2.at[dirn, h - 1]
            return pltpu.make_async_remote_copy(
                src_ref=src,
                dst_ref=rbufB2.at[dirn, h],
                send_sem=sB2_s.at[dirn, h],
                recv_sem=sB2_r.at[dirn, h],
                device_id=(dst_pl[dirn],),
                device_id_type=pl.DeviceIdType.MESH,
            )

        for g in range(NZ):
            for dirn in (0, 1):
                rd = a1_make(dirn, 0, g)
                rd.start()
                rdmas[(1, dirn, 0, g)] = rd
        for s in range(BZF):
            for dirn in (0, 1):
                rd = b1_make(dirn, 0, s)
                rd.start()
                rdmas[(3, dirn, 0, s)] = rd

        def b2_process(h):
            for dirn in (0, 1):
                rdmas[(4, dirn, h)].wait_recv()
                rc = lax.rem(r + sign[dirn] * (2 + h) + 32, 8)
                qc = r_to_q(rc)
                if h < PLANE - 2:
                    rbufB2[dirn, h] = (
                        rbufB2[dirn, h]
                        + rbufB1[dirn, NZ - 2, pl.ds(qc * m_per, m_per)]
                    )
                    rd = b2_make(dirn, h + 1)
                    rd.start()
                    rdmas[(4, dirn, h + 1)] = rd
                else:
                    y = (
                        rbufB2[dirn, h]
                        + rbufB1[dirn, NZ - 2, pl.ds(qc * m_per, m_per)]
                    )
                    yc = jnp.clip(y, -60.0, 60.0)
                    out_ref[:, pl.ds(bcol0[dirn], bhalf)] = (
                        y / (1.0 + jnp.exp(-yc))
                    )

        for h in range(PLANE - 1):
            if h == PLANE - 2:
                b1_process(2)
                for dirn in (0, 1):
                    rd = b2_make(dirn, 0)
                    rd.start()
                    rdmas[(4, dirn, 0)] = rd
            for g in range(NZ):
                for dirn in (0, 1):
                    rdmas[(1, dirn, h, g)].wait_recv()
                    rc = lax.rem(r + sign[dirn] * (2 + h) + 32, 8)
                    b = zb_of(dirn, g) * PLANE + r_to_q(rc)
                    rbufA1[dirn, h, g] = (
                        rbufA1[dirn, h, g]
                        + part_ref[pl.ds(b * m_per, m_per),
                                   pl.ds(acol0[dirn], ahalf)]
                    )
                    if h < PLANE - 2:
                        rd = a1_make(dirn, h + 1, g)
                        rd.start()
                        rdmas[(1, dirn, h + 1, g)] = rd
                    else:
                        if g == 0:
                            rd = a2_make(dirn, 0)
                            rd.start()
                            rdmas[(2, dirn, 0)] = rd
                        elif g < NZ - 1:
                            rdmas[(2, dirn, g - 1)].wait_recv()
                            rbufA2[dirn, g - 1] = (
                                rbufA2[dirn, g - 1]
                                + rbufA1[dirn, PLANE - 2, g]
                            )
                            rd = a2_make(dirn, g)
                            rd.start()
                            rdmas[(2, dirn, g)] = rd
                        else:
                            rdmas[(2, dirn, g - 1)].wait_recv()
                            y = (
                                rbufA2[dirn, g - 1]
                                + rbufA1[dirn, PLANE - 2, g]
                            )
                            yc = jnp.clip(y, -60.0, 60.0)
                            out_ref[:, pl.ds(acol0[dirn], ahalf)] = (
                                y / (1.0 + jnp.exp(-yc))
                            )
                if h == PLANE - 2 and g < NZ - 1:
                    b2_process(g)
            if h == 2:
                b1_process(0)
            elif h == 4:
                b1_process(1)

        for h in range(NZ - 1, PLANE - 1):
            b2_process(h)

        for key in rdmas:
            rdmas[key].wait_send()

    return pl.pallas_call(
        body,
        out_shape=jax.ShapeDtypeStruct((m_per, n), jnp.float32),
        in_specs=[
            pl.BlockSpec(memory_space=pltpu.VMEM),
            pl.BlockSpec(memory_space=pltpu.VMEM),
        ],
        out_specs=pl.BlockSpec(memory_space=pltpu.VMEM),
        scratch_shapes=[
            pltpu.VMEM((m, n), jnp.float32),
            pltpu.VMEM((2, PLANE - 1, NZ, m_per, ahalf), jnp.float32),
            pltpu.VMEM((2, NZ - 1, m_per, ahalf), jnp.float32),
            pltpu.VMEM((2, NZ - 1, mz, bhalf), jnp.float32),
            pltpu.VMEM((2, PLANE - 1, m_per, bhalf), jnp.float32),
            pltpu.SemaphoreType.DMA((2, PLANE - 1, NZ)),
            pltpu.SemaphoreType.DMA((2, PLANE - 1, NZ)),
            pltpu.SemaphoreType.DMA((2, NZ - 1)),
            pltpu.SemaphoreType.DMA((2, NZ - 1)),
            pltpu.SemaphoreType.DMA((2, NZ - 1, BZF)),
            pltpu.SemaphoreType.DMA((2, NZ - 1, BZF)),
            pltpu.SemaphoreType.DMA((2, PLANE - 1)),
            pltpu.SemaphoreType.DMA((2, PLANE - 1)),
        ],
        compiler_params=pltpu.CompilerParams(
            collective_id=0, vmem_limit_bytes=64 * 1024 * 1024
        ),
    )(x, w_mat)


# device time: 95781 ns/iter; 1.1234x vs baseline; 1.0275x over previous
import jax
import jax.numpy as jnp
from jax import lax
from jax.experimental import pallas as pl
from jax.experimental.pallas import tpu as pltpu

N_DEV = 32
PLANE = 8
NZ = 4
ACOL = 1536
BZF = 4


def kernel(x, w_mat):
    m, k_per = x.shape
    _, n = w_mat.shape
    m_per = m // N_DEV
    ahalf = ACOL // 2
    bcol = n - ACOL
    bhalf = bcol // 2
    mz = m // NZ
    bzw = mz // BZF

    def body(x_ref, w_ref, out_ref, part_ref, rbufA1, rbufA2, rbufB1,
             rbufB2, sA1_s, sA1_r, sA2_s, sA2_r, sB1_s, sB1_r,
             sB2_s, sB2_r):
        p = lax.axis_index("i")
        z = lax.div(p, PLANE)
        q = lax.rem(p, PLANE)

        def q_to_r(qq):
            yy = lax.div(qq, 2)
            xx = lax.rem(qq + yy, 2)
            return jnp.where(xx == 1, 1 + yy, lax.rem(8 - yy, 8))

        def r_to_q(rr):
            xx = jnp.where((rr >= 1) & (rr <= 4), 1, 0)
            yy = jnp.where(xx == 1, rr - 1, lax.rem(8 - rr, 8))
            return 2 * yy + lax.rem(xx + yy, 2)

        r = q_to_r(q)

        succ = z * PLANE + r_to_q(lax.rem(r + 1, 8))
        pred = z * PLANE + r_to_q(lax.rem(r + 7, 8))
        up = lax.rem(z + 1, NZ) * PLANE + q
        down = lax.rem(z + 3, NZ) * PLANE + q

        part_ref[:, :] = jnp.dot(
            x_ref[:, :], w_ref[:, :], preferred_element_type=jnp.float32
        )

        barrier_sem = pltpu.get_barrier_semaphore()
        for nbr in (succ, pred, up, down):
            pl.semaphore_signal(
                barrier_sem, inc=1,
                device_id=(nbr,), device_id_type=pl.DeviceIdType.MESH,
            )
        pl.semaphore_wait(barrier_sem, 4)

        acol0 = (0, ahalf)
        bcol0 = (ACOL, ACOL + bhalf)
        sign = (-1, 1)
        dst_pl = (succ, pred)
        dst_z = (up, down)

        rdmas = {}

        def zb_of(dirn, g):
            if dirn == 0:
                return lax.rem(z + 3 - g + NZ, NZ)
            return lax.rem(z + 1 + g, NZ)

        def a1_make(dirn, h, g):
            if h == 0:
                rc = lax.rem(r + sign[dirn] + 16, 8)
                b = zb_of(dirn, g) * PLANE + r_to_q(rc)
                src = part_ref.at[pl.ds(b * m_per, m_per),
                                  pl.ds(acol0[dirn], ahalf)]
            else:
                src = rbufA1.at[dirn, h - 1, g]
            return pltpu.make_async_remote_copy(
                src_ref=src,
                dst_ref=rbufA1.at[dirn, h, g],
                send_sem=sA1_s.at[dirn, h, g],
                recv_sem=sA1_r.at[dirn, h, g],
                device_id=(dst_pl[dirn],),
                device_id_type=pl.DeviceIdType.MESH,
            )

        def a2_make(dirn, h):
            if h == 0:
                src = rbufA1.at[dirn, PLANE - 2, 0]
            else:
                src = rbufA2.at[dirn, h - 1]
            return pltpu.make_async_remote_copy(
                src_ref=src,
                dst_ref=rbufA2.at[dirn, h],
                send_sem=sA2_s.at[dirn, h],
                recv_sem=sA2_r.at[dirn, h],
                device_id=(dst_z[dirn],),
                device_id_type=pl.DeviceIdType.MESH,
            )

        def b1_make(dirn, h, s):
            rds = pl.ds(s * bzw, bzw)
            if h == 0:
                zc = lax.rem(z + sign[dirn] + 8, NZ)
                src = part_ref.at[pl.ds(zc * mz + s * bzw, bzw),
                                  pl.ds(bcol0[dirn], bhalf)]
            else:
                src = rbufB1.at[dirn, h - 1, rds]
            return pltpu.make_async_remote_copy(
                src_ref=src,
                dst_ref=rbufB1.at[dirn, h, rds],
                send_sem=sB1_s.at[dirn, h, s],
                recv_sem=sB1_r.at[dirn, h, s],
                device_id=(dst_z[dirn],),
                device_id_type=pl.DeviceIdType.MESH,
            )

        def b1_process(h):
            for s in range(BZF):
                for dirn in (0, 1):
                    rdmas[(3, dirn, h, s)].wait_recv()
                    zc = lax.rem(z + sign[dirn] * (2 + h) + 8, NZ)
                    rds = pl.ds(s * bzw, bzw)
                    rbufB1[dirn, h, rds] = (
                        rbufB1[dirn, h, rds]
                        + part_ref[pl.ds(zc * mz + s * bzw, bzw),
                                   pl.ds(bcol0[dirn], bhalf)]
                    )
                    if h < NZ - 2:
                        rd = b1_make(dirn, h + 1, s)
                        rd.start()
                        rdmas[(3, dirn, h + 1, s)] = rd

        def b2_make(dirn, h):
            if h == 0:
                rc = lax.rem(r + sign[dirn] + 16, 8)
                src = rbufB1.at[dirn, NZ - 2,
                                pl.ds(r_to_q(rc) * m_per, m_per)]
            else:
                src = rbufB2.at[dirn, h - 1]
            return pltpu.make_async_remote_copy(
                src_ref=src,
                dst_ref=rbufB2.at[dirn, h],
                send_sem=sB2_s.at[dirn, h],
                recv_sem=sB2_r.at[dirn, h],
                device_id=(dst_pl[dirn],),
                device_id_type=pl.DeviceIdType.MESH,
            )

        for g in range(NZ):
            for dirn in (0, 1):
                rd = a1_make(dirn, 0, g)
                rd.start()
                rdmas[(1, dirn, 0, g)] = rd
        for s in range(BZF):
            for dirn in (0, 1):
                rd = b1_make(dirn, 0, s)
                rd.start()
                rdmas[(3, dirn, 0, s)] = rd

        def b2_process(h):
            for dirn in (0, 1):
                rdmas[(4, dirn, h)].wait_recv()
                rc = lax.rem(r + sign[dirn] * (2 + h) + 32, 8)
                qc = r_to_q(rc)
                if h < PLANE - 2:
                    rbufB2[dirn, h] = (
                        rbufB2[dirn, h]
                        + rbufB1[dirn, NZ - 2, pl.ds(qc * m_per, m_per)]
                    )
                    rd = b2_make(dirn, h + 1)
                    rd.start()
                    rdmas[(4, dirn, h + 1)] = rd
                else:
                    y = (
                        rbufB2[dirn, h]
                        + rbufB1[dirn, NZ - 2, pl.ds(qc * m_per, m_per)]
                    )
                    yc = jnp.clip(y, -60.0, 60.0)
                    out_ref[:, pl.ds(bcol0[dirn], bhalf)] = (
                        y / (1.0 + jnp.exp(-yc))
                    )

        for h in range(PLANE - 1):
            if h == PLANE - 2:
                b1_process(2)
                for dirn in (0, 1):
                    rd = b2_make(dirn, 0)
                    rd.start()
                    rdmas[(4, dirn, 0)] = rd
            for g in range(NZ):
                for dirn in (0, 1):
                    rdmas[(1, dirn, h, g)].wait_recv()
                    rc = lax.rem(r + sign[dirn] * (2 + h) + 32, 8)
                    b = zb_of(dirn, g) * PLANE + r_to_q(rc)
                    rbufA1[dirn, h, g] = (
                        rbufA1[dirn, h, g]
                        + part_ref[pl.ds(b * m_per, m_per),
                                   pl.ds(acol0[dirn], ahalf)]
                    )
                    if h < PLANE - 2:
                        rd = a1_make(dirn, h + 1, g)
                        rd.start()
                        rdmas[(1, dirn, h + 1, g)] = rd
                    else:
                        if g == 0:
                            rd = a2_make(dirn, 0)
                            rd.start()
                            rdmas[(2, dirn, 0)] = rd
                        elif g < NZ - 1:
                            rdmas[(2, dirn, g - 1)].wait_recv()
                            rbufA2[dirn, g - 1] = (
                                rbufA2[dirn, g - 1]
                                + rbufA1[dirn, PLANE - 2, g]
                            )
                            rd = a2_make(dirn, g)
                            rd.start()
                            rdmas[(2, dirn, g)] = rd
                        else:
                            rdmas[(2, dirn, g - 1)].wait_recv()
                            y = (
                                rbufA2[dirn, g - 1]
                                + rbufA1[dirn, PLANE - 2, g]
                            )
                            yc = jnp.clip(y, -60.0, 60.0)
                            out_ref[:, pl.ds(acol0[dirn], ahalf)] = (
                                y / (1.0 + jnp.exp(-yc))
                            )
                if h == PLANE - 2 and g < NZ - 1:
                    b2_process(g)
            if h == 2:
                b1_process(0)
            elif h == 4:
                b1_process(1)

        for h in range(NZ - 1, PLANE - 1):
            b2_process(h)

        for key in rdmas:
            rdmas[key].wait_send()

    return pl.pallas_call(
        body,
        out_shape=jax.ShapeDtypeStruct((m_per, n), jnp.float32),
        in_specs=[
            pl.BlockSpec(memory_space=pltpu.VMEM),
            pl.BlockSpec(memory_space=pltpu.VMEM),
        ],
        out_specs=pl.BlockSpec(memory_space=pltpu.VMEM),
        scratch_shapes=[
            pltpu.VMEM((m, n), jnp.float32),
            pltpu.VMEM((2, PLANE - 1, NZ, m_per, ahalf), jnp.float32),
            pltpu.VMEM((2, NZ - 1, m_per, ahalf), jnp.float32),
            pltpu.VMEM((2, NZ - 1, mz, bhalf), jnp.float32),
            pltpu.VMEM((2, PLANE - 1, m_per, bhalf), jnp.float32),
            pltpu.SemaphoreType.DMA((2, PLANE - 1, NZ)),
            pltpu.SemaphoreType.DMA((2, PLANE - 1, NZ)),
            pltpu.SemaphoreType.DMA((2, NZ - 1)),
            pltpu.SemaphoreType.DMA((2, NZ - 1)),
            pltpu.SemaphoreType.DMA((2, NZ - 1, BZF)),
            pltpu.SemaphoreType.DMA((2, NZ - 1, BZF)),
            pltpu.SemaphoreType.DMA((2, PLANE - 1)),
            pltpu.SemaphoreType.DMA((2, PLANE - 1)),
        ],
        compiler_params=pltpu.CompilerParams(
            collective_id=0, vmem_limit_bytes=64 * 1024 * 1024
        ),
    )(x, w_mat)
